# Initial kernel scaffold; baseline (speedup 1.0000x reference)
import jax
import jax.numpy as jnp
from jax.experimental import pallas as pl


def kernel(x, Ws, att_src, att_dst, bias_gat, W1, b1, W2, b2, edge_src, edge_dst, edge_offsets):
    raise NotImplementedError("write your pallas kernel here")


# speedup vs baseline: 24.2396x; 24.2396x over previous
import functools
import jax
import jax.numpy as jnp
from jax.experimental import pallas as pl
from jax.experimental.pallas import tpu as pltpu

N = 10000
D = 128
EXT = 144
NB = 2000
NBLK = N // NB


def _tc1_body(x_ref, w_ref, as_ref, ad_ref, hext_ref, asn_ref, adn_ref, es_ref):
    h = jnp.dot(x_ref[...], w_ref[0], preferred_element_type=jnp.float32)
    ones = jnp.ones((NB, 1), jnp.float32)
    pad = jnp.zeros((NB, EXT - D - 1), jnp.float32)
    hext_ref[0] = jnp.concatenate([h, ones, pad], axis=1)
    asn = jnp.sum(h * as_ref[0], axis=1, keepdims=True)
    adn = jnp.sum(h * ad_ref[0], axis=1, keepdims=True)
    asn_ref[0] = asn
    adn_ref[0] = adn
    e_self = asn + adn
    e_self = jnp.where(e_self >= 0, e_self, 0.2 * e_self)
    es_ref[0] = jnp.exp(e_self)


def _phase1(x, Ws, att_src, att_dst):
    n_agg = Ws.shape[0]
    grid = (NBLK, n_agg)
    return pl.pallas_call(
        _tc1_body,
        grid=grid,
        in_specs=[
            pl.BlockSpec((NB, D), lambda b, i: (b, 0)),
            pl.BlockSpec((1, D, D), lambda b, i: (i, 0, 0)),
            pl.BlockSpec((1, 1, D), lambda b, i: (i, 0, 0)),
            pl.BlockSpec((1, 1, D), lambda b, i: (i, 0, 0)),
        ],
        out_specs=[
            pl.BlockSpec((1, NB, EXT), lambda b, i: (i, b, 0)),
            pl.BlockSpec((1, NB, 1), lambda b, i: (i, b, 0)),
            pl.BlockSpec((1, NB, 1), lambda b, i: (i, b, 0)),
            pl.BlockSpec((1, NB, 1), lambda b, i: (i, b, 0)),
        ],
        out_shape=[
            jax.ShapeDtypeStruct((n_agg, N, EXT), jnp.float32),
            jax.ShapeDtypeStruct((n_agg, N, 1), jnp.float32),
            jax.ShapeDtypeStruct((n_agg, N, 1), jnp.float32),
            jax.ShapeDtypeStruct((n_agg, N, 1), jnp.float32),
        ],
        compiler_params=pltpu.CompilerParams(
            dimension_semantics=("parallel", "arbitrary")),
    )(x, Ws, att_src.reshape(n_agg, 1, D), att_dst.reshape(n_agg, 1, D))


def _tc2_body(x_ref, ws_ref, es_ref, acc_ref, bias_ref, w1x_ref, w1r_ref,
              b1_ref, w2_ref, b2_ref, y_ref, *, n_agg):
    i = pl.program_id(1)

    @pl.when(i == 0)
    def _():
        y_ref[...] = jnp.dot(x_ref[...], w1x_ref[...],
                             preferred_element_type=jnp.float32)

    h = jnp.dot(x_ref[...], ws_ref[0], preferred_element_type=jnp.float32)
    es = es_ref[0]
    acc = jnp.sum(acc_ref[:, 0], axis=0)
    num = acc[:, :D] + es * h
    den = acc[:, D:D + 1] + es
    out_i = num / den + bias_ref[0]
    y_ref[...] += jnp.dot(out_i, w1r_ref[0], preferred_element_type=jnp.float32)

    @pl.when(i == n_agg - 1)
    def _():
        mid = jnp.tanh(y_ref[...] + b1_ref[...])
        y_ref[...] = jnp.dot(mid, w2_ref[...],
                             preferred_element_type=jnp.float32) + b2_ref[...]


def _phase3(x, Ws, es, acc, bias_gat, W1, b1, W2, b2):
    n_agg = Ws.shape[0]
    npart = acc.shape[0]
    W1x = W1[:D]
    W1r = W1[D:].reshape(n_agg, D, D)
    grid = (NBLK, n_agg)
    return pl.pallas_call(
        functools.partial(_tc2_body, n_agg=n_agg),
        grid=grid,
        in_specs=[
            pl.BlockSpec((NB, D), lambda b, i: (b, 0)),
            pl.BlockSpec((1, D, D), lambda b, i: (i, 0, 0)),
            pl.BlockSpec((1, NB, 1), lambda b, i: (i, b, 0)),
            pl.BlockSpec((npart, 1, NB, EXT), lambda b, i: (0, i, b, 0)),
            pl.BlockSpec((1, 1, D), lambda b, i: (i, 0, 0)),
            pl.BlockSpec((D, D), lambda b, i: (0, 0)),
            pl.BlockSpec((1, D, D), lambda b, i: (i, 0, 0)),
            pl.BlockSpec((1, D), lambda b, i: (0, 0)),
            pl.BlockSpec((D, D), lambda b, i: (0, 0)),
            pl.BlockSpec((1, D), lambda b, i: (0, 0)),
        ],
        out_specs=pl.BlockSpec((NB, D), lambda b, i: (b, 0)),
        out_shape=jax.ShapeDtypeStruct((N, D), jnp.float32),
        compiler_params=pltpu.CompilerParams(
            dimension_semantics=("parallel", "arbitrary")),
    )(x, Ws, es, acc, bias_gat.reshape(n_agg, 1, D), W1x, W1r,
      b1.reshape(1, D), W2, b2.reshape(1, D))


def _edge_phase(Hext, asn, adn, edge_src, edge_dst, edge_offsets, n_agg):
    T = edge_src.shape[0]
    idx = jnp.arange(T, dtype=jnp.int32)
    agg = jnp.searchsorted(edge_offsets.astype(jnp.int32), idx, side="right") - 1
    e = asn[agg, edge_src, 0] + adn[agg, edge_dst, 0]
    e = jnp.where(e >= 0, e, 0.2 * e)
    w = jnp.exp(e)
    rows = Hext.reshape(n_agg * N, EXT)[agg * N + edge_src]
    acc = jax.ops.segment_sum(w[:, None] * rows, agg * N + edge_dst,
                              num_segments=n_agg * N)
    return acc.reshape(1, n_agg, N, EXT)


def kernel(x, Ws, att_src, att_dst, bias_gat, W1, b1, W2, b2,
           edge_src, edge_dst, edge_offsets):
    n_agg = Ws.shape[0]
    Hext, asn, adn, es = _phase1(x, Ws, att_src, att_dst)
    acc = _edge_phase(Hext, asn, adn, edge_src.astype(jnp.int32),
                      edge_dst.astype(jnp.int32), edge_offsets, n_agg)
    return _phase3(x, Ws, es, acc, bias_gat, W1, b1, W2, b2)


# speedup vs baseline: 320.7447x; 13.2322x over previous
import functools
import jax
import jax.numpy as jnp
from jax import lax
from jax.experimental import pallas as pl
from jax.experimental.pallas import tpu as pltpu
from jax.experimental.pallas import tpu_sc as plsc

N = 10000
D = 128
NB = 2000
NBLK = N // NB

EB = 64
ROWS = 10112
STRIPE = ROWS // 16
DROWS = 128


def _tc1_body(x_ref, w_ref, as_ref, ad_ref, h_ref, asn_ref, adn_ref, es_ref):
    h = jnp.dot(x_ref[...], w_ref[0], preferred_element_type=jnp.float32)
    h_ref[0] = h
    asn = jnp.sum(h * as_ref[0], axis=1, keepdims=True)
    adn = jnp.sum(h * ad_ref[0], axis=1, keepdims=True)
    asn_ref[0] = asn
    adn_ref[0] = adn
    e_self = asn + adn
    e_self = jnp.where(e_self >= 0, e_self, 0.2 * e_self)
    es_ref[0] = jnp.exp(e_self)


def _phase1(x, Ws, att_src, att_dst):
    n_agg = Ws.shape[0]
    grid = (NBLK, n_agg)
    return pl.pallas_call(
        _tc1_body,
        grid=grid,
        in_specs=[
            pl.BlockSpec((NB, D), lambda b, i: (b, 0)),
            pl.BlockSpec((1, D, D), lambda b, i: (i, 0, 0)),
            pl.BlockSpec((1, 1, D), lambda b, i: (i, 0, 0)),
            pl.BlockSpec((1, 1, D), lambda b, i: (i, 0, 0)),
        ],
        out_specs=[
            pl.BlockSpec((1, NB, D), lambda b, i: (i, b, 0)),
            pl.BlockSpec((1, NB, 1), lambda b, i: (i, b, 0)),
            pl.BlockSpec((1, NB, 1), lambda b, i: (i, b, 0)),
            pl.BlockSpec((1, NB, 1), lambda b, i: (i, b, 0)),
        ],
        out_shape=[
            jax.ShapeDtypeStruct((n_agg, N, D), jnp.float32),
            jax.ShapeDtypeStruct((n_agg, N, 1), jnp.float32),
            jax.ShapeDtypeStruct((n_agg, N, 1), jnp.float32),
            jax.ShapeDtypeStruct((n_agg, N, 1), jnp.float32),
        ],
        compiler_params=pltpu.CompilerParams(
            dimension_semantics=("parallel", "arbitrary")),
    )(x, Ws, att_src.reshape(n_agg, 1, D), att_dst.reshape(n_agg, 1, D))


def _sc_body(n_agg, h_ref, esrc_ref, edst_ref, desc_ref,
             zeros_ref, num_out, den_out, asn_t, adn_t, desc_v, src_v, dst_v,
             didx_v, dden_v, lane_v, w_v, rows_v, wrows_v, acc, dacc, sem):
    c = lax.axis_index("c")
    s = lax.axis_index("s")
    pltpu.sync_copy(desc_ref.at[c, s], desc_v)
    pltpu.sync_copy(zeros_ref.at[pl.ds(0, EB)], wrows_v)
    iota = jnp.arange(16, dtype=jnp.int32)
    zvec = jnp.zeros((16,), jnp.float32)
    asn0 = n_agg * N
    adn0 = n_agg * N + n_agg * 80

    def agg_body(i, _):
        v = desc_v[pl.ds(i * 16, 16)]
        lo_s = v[0]
        hi_s = v[1]
        k0_s = v[2]
        nk_s = v[3]
        pltpu.sync_copy(h_ref.at[pl.ds(asn0 + i * 80, 80)], asn_t)
        pltpu.sync_copy(h_ref.at[pl.ds(adn0 + i * 80, 80)], adn_t)
        pltpu.sync_copy(zeros_ref.at[pl.ds(s * STRIPE, STRIPE)],
                        acc.at[pl.ds(s * STRIPE, STRIPE)])
        pltpu.sync_copy(zeros_ref.at[pl.ds(0, 8)], dacc.at[pl.ds(s * 8, 8)])
        plsc.subcore_barrier()

        def batch_body(j, _):
            e0 = (k0_s + 32 * j) * EB
            pltpu.sync_copy(esrc_ref.at[pl.ds(e0, EB)], src_v)
            pltpu.sync_copy(edst_ref.at[pl.ds(e0, EB)], dst_v)
            for g in range(EB // 16):
                sl = pl.ds(g * 16, 16)
                sg = src_v[sl]
                dg = dst_v[sl]
                a = plsc.load_gather(asn_t, [sg >> 7, sg & 127])
                b = plsc.load_gather(adn_t, [dg >> 7, dg & 127])
                e = a + b
                e = jnp.where(e >= 0, e, 0.2 * e)
                w = jnp.exp(e)
                eix = e0 + g * 16 + iota
                valid = (eix >= lo_s) & (eix < hi_s)
                w = jnp.where(valid, w, 0.0)
                lanes = dg & 127
                w_v[sl] = w
                didx_v[sl] = jnp.where(valid, dg, N)
                dden_v[sl] = jnp.where(valid, dg >> 7, 120)
                lane_v[sl] = lanes
                src_v[sl] = sg + i * N
                plsc.store_scatter(wrows_v, [g * 16 + iota, lanes], w)
            pltpu.async_copy(h_ref.at[src_v], rows_v, sem).wait()

            def row_body(r, _):
                wb = plsc.load_gather(w_v, [jnp.zeros((16,), jnp.int32) + r])
                for cc in range(D // 16):
                    csl = pl.ds(cc * 16, 16)
                    rows_v[r, csl] = rows_v[r, csl] * wb
                return 0

            lax.fori_loop(0, EB, row_body, 0)
            pltpu.sync_copy(rows_v, acc.at[didx_v], add=True)
            pltpu.sync_copy(wrows_v, dacc.at[dden_v], add=True)
            for g in range(EB // 16):
                sl = pl.ds(g * 16, 16)
                plsc.store_scatter(wrows_v, [g * 16 + iota, lane_v[sl]], zvec)
            return 0

        lax.fori_loop(0, nk_s, batch_body, 0)
        plsc.subcore_barrier()
        pltpu.sync_copy(acc.at[pl.ds(s * STRIPE, STRIPE)],
                        num_out.at[c, i, pl.ds(s * STRIPE, STRIPE)])
        pltpu.sync_copy(dacc.at[pl.ds(s * 8, 8)],
                        den_out.at[c, i, pl.ds(s * 8, 8)])
        return 0

    lax.fori_loop(0, n_agg, agg_body, 0)


def _edge_phase(H, asn, adn, edge_src, edge_dst, edge_offsets, n_agg):
    T = edge_src.shape[0]
    nbatch = -(-T // EB)
    tpad = nbatch * EB
    pad = jnp.zeros((tpad - T,), jnp.int32)
    esrc = jnp.concatenate([edge_src, pad])
    edst = jnp.concatenate([edge_dst, pad])
    off = edge_offsets.astype(jnp.int32)
    lo = off[:-1]
    hi = off[1:]
    b0 = lo // EB
    b1 = -(-hi // EB)
    cs = jnp.arange(2, dtype=jnp.int32)[:, None, None]
    ss = jnp.arange(16, dtype=jnp.int32)[None, :, None]
    k0 = b0[None, None, :] + cs + 2 * ss
    nk = jnp.maximum(0, -(-(b1[None, None, :] - k0) // 32))
    desc = jnp.zeros((2, 16, n_agg, 16), jnp.int32)
    desc = desc.at[..., 0].set(jnp.broadcast_to(lo, (2, 16, n_agg)))
    desc = desc.at[..., 1].set(jnp.broadcast_to(hi, (2, 16, n_agg)))
    desc = desc.at[..., 2].set(k0)
    desc = desc.at[..., 3].set(nk)
    desc = desc.reshape(2, 16, n_agg * 16)
    zeros_h = jnp.zeros((ROWS, D), jnp.float32)

    mesh = plsc.VectorSubcoreMesh(core_axis_name="c", subcore_axis_name="s")
    f = pl.kernel(
        functools.partial(_sc_body, n_agg),
        out_type=[
            jax.ShapeDtypeStruct((2, n_agg, ROWS, D), jnp.float32),
            jax.ShapeDtypeStruct((2, n_agg, DROWS, D), jnp.float32),
        ],
        mesh=mesh,
        compiler_params=pltpu.CompilerParams(needs_layout_passes=False),
        scratch_types=[
            pltpu.VMEM((80, D), jnp.float32),
            pltpu.VMEM((80, D), jnp.float32),
            pltpu.VMEM((n_agg * 16,), jnp.int32),
            pltpu.VMEM((EB,), jnp.int32),
            pltpu.VMEM((EB,), jnp.int32),
            pltpu.VMEM((EB,), jnp.int32),
            pltpu.VMEM((EB,), jnp.int32),
            pltpu.VMEM((EB,), jnp.int32),
            pltpu.VMEM((EB,), jnp.float32),
            pltpu.VMEM((EB, D), jnp.float32),
            pltpu.VMEM((EB, D), jnp.float32),
            pltpu.VMEM_SHARED((ROWS, D), jnp.float32),
            pltpu.VMEM_SHARED((DROWS, D), jnp.float32),
            pltpu.SemaphoreType.DMA,
        ],
    )
    def pack(a):
        a = a.reshape(n_agg, N)
        a = jnp.concatenate(
            [a, jnp.zeros((n_agg, 80 * D - N), jnp.float32)], axis=1)
        return a.reshape(n_agg * 80, D)

    haug = jnp.concatenate([H.reshape(n_agg * N, D), pack(asn), pack(adn)])
    num, den = f(haug, esrc, edst, desc, zeros_h)
    den = den.reshape(2, n_agg, DROWS * D)[:, :, :N].reshape(2, n_agg, N, 1)
    return num, den


def _tc2_body(x_ref, ws_ref, es_ref, num_ref, den_ref, bias_ref, w1x_ref,
              w1r_ref, b1_ref, w2_ref, b2_ref, y_ref, *, n_agg):
    i = pl.program_id(1)

    @pl.when(i == 0)
    def _():
        y_ref[...] = jnp.dot(x_ref[...], w1x_ref[...],
                             preferred_element_type=jnp.float32)

    h = jnp.dot(x_ref[...], ws_ref[0], preferred_element_type=jnp.float32)
    es = es_ref[0]
    num = jnp.sum(num_ref[:, 0], axis=0) + es * h
    den = jnp.sum(den_ref[:, 0], axis=0) + es
    out_i = num / den + bias_ref[0]
    y_ref[...] += jnp.dot(out_i, w1r_ref[0], preferred_element_type=jnp.float32)

    @pl.when(i == n_agg - 1)
    def _():
        mid = jnp.tanh(y_ref[...] + b1_ref[...])
        y_ref[...] = jnp.dot(mid, w2_ref[...],
                             preferred_element_type=jnp.float32) + b2_ref[...]


def _phase3(x, Ws, es, num, den, bias_gat, W1, b1, W2, b2):
    n_agg = Ws.shape[0]
    W1x = W1[:D]
    W1r = W1[D:].reshape(n_agg, D, D)
    grid = (NBLK, n_agg)
    return pl.pallas_call(
        functools.partial(_tc2_body, n_agg=n_agg),
        grid=grid,
        in_specs=[
            pl.BlockSpec((NB, D), lambda b, i: (b, 0)),
            pl.BlockSpec((1, D, D), lambda b, i: (i, 0, 0)),
            pl.BlockSpec((1, NB, 1), lambda b, i: (i, b, 0)),
            pl.BlockSpec((2, 1, NB, D), lambda b, i: (0, i, b, 0)),
            pl.BlockSpec((2, 1, NB, 1), lambda b, i: (0, i, b, 0)),
            pl.BlockSpec((1, 1, D), lambda b, i: (i, 0, 0)),
            pl.BlockSpec((D, D), lambda b, i: (0, 0)),
            pl.BlockSpec((1, D, D), lambda b, i: (i, 0, 0)),
            pl.BlockSpec((1, D), lambda b, i: (0, 0)),
            pl.BlockSpec((D, D), lambda b, i: (0, 0)),
            pl.BlockSpec((1, D), lambda b, i: (0, 0)),
        ],
        out_specs=pl.BlockSpec((NB, D), lambda b, i: (b, 0)),
        out_shape=jax.ShapeDtypeStruct((N, D), jnp.float32),
        compiler_params=pltpu.CompilerParams(
            dimension_semantics=("parallel", "arbitrary")),
    )(x, Ws, es, num, den, bias_gat.reshape(n_agg, 1, D), W1x, W1r,
      b1.reshape(1, D), W2, b2.reshape(1, D))


def kernel(x, Ws, att_src, att_dst, bias_gat, W1, b1, W2, b2,
           edge_src, edge_dst, edge_offsets):
    n_agg = Ws.shape[0]
    H, asn, adn, es = _phase1(x, Ws, att_src, att_dst)
    num, den = _edge_phase(H, asn, adn, edge_src.astype(jnp.int32),
                           edge_dst.astype(jnp.int32), edge_offsets, n_agg)
    return _phase3(x, Ws, es, num, den, bias_gat, W1, b1, W2, b2)


# speedup vs baseline: 361.1022x; 1.1258x over previous
import functools
import jax
import jax.numpy as jnp
from jax import lax
from jax.experimental import pallas as pl
from jax.experimental.pallas import tpu as pltpu
from jax.experimental.pallas import tpu_sc as plsc

N = 10000
NPAD = 10240
D = 128
NB = 2048
NBLK = NPAD // NB

EB = 64
ROWS = NPAD
STRIPE = ROWS // 16
TRASH = 10200
DROWS = 128


def _tc1_body(x_ref, w_ref, as_ref, ad_ref, h_ref, asn_ref, adn_ref):
    h = jnp.dot(x_ref[...], w_ref[0], preferred_element_type=jnp.float32)
    h_ref[0] = h
    asn_ref[0] = lax.dot_general(as_ref[0], h, (((1,), (1,)), ((), ())),
                                 preferred_element_type=jnp.float32)
    adn_ref[0] = lax.dot_general(ad_ref[0], h, (((1,), (1,)), ((), ())),
                                 preferred_element_type=jnp.float32)


def _phase1(x, Ws, att_src, att_dst):
    n_agg = Ws.shape[0]
    grid = (NBLK, n_agg)
    return pl.pallas_call(
        _tc1_body,
        grid=grid,
        in_specs=[
            pl.BlockSpec((NB, D), lambda b, i: (b, 0)),
            pl.BlockSpec((1, D, D), lambda b, i: (i, 0, 0)),
            pl.BlockSpec((1, 1, D), lambda b, i: (i, 0, 0)),
            pl.BlockSpec((1, 1, D), lambda b, i: (i, 0, 0)),
        ],
        out_specs=[
            pl.BlockSpec((1, NB, D), lambda b, i: (i, b, 0)),
            pl.BlockSpec((1, 1, NB), lambda b, i: (i, 0, b)),
            pl.BlockSpec((1, 1, NB), lambda b, i: (i, 0, b)),
        ],
        out_shape=[
            jax.ShapeDtypeStruct((n_agg, NPAD, D), jnp.float32),
            jax.ShapeDtypeStruct((n_agg, 1, NPAD), jnp.float32),
            jax.ShapeDtypeStruct((n_agg, 1, NPAD), jnp.float32),
        ],
        compiler_params=pltpu.CompilerParams(
            dimension_semantics=("parallel", "arbitrary")),
    )(x, Ws, att_src.reshape(n_agg, 1, D), att_dst.reshape(n_agg, 1, D))


def _sc_body(n_agg, h_ref, esrc_ref, edst_ref, desc_ref,
             zeros_ref, num_out, den_out, asn_t, adn_t, desc_v, src_v, dst_v,
             didx_v, dden_v, lane_v, w_v, rows_v, wrows_v, acc, dacc, sem):
    c = lax.axis_index("c")
    s = lax.axis_index("s")
    pltpu.sync_copy(desc_ref.at[c, s], desc_v)
    pltpu.sync_copy(zeros_ref.at[pl.ds(0, EB)], wrows_v)
    iota = jnp.arange(16, dtype=jnp.int32)
    zvec = jnp.zeros((16,), jnp.float32)
    asn0 = n_agg * NPAD
    adn0 = n_agg * NPAD + n_agg * 80

    def agg_body(i, _):
        v = desc_v[pl.ds(i * 16, 16)]
        lo_s = v[0]
        hi_s = v[1]
        k0_s = v[2]
        nk_s = v[3]
        pltpu.sync_copy(h_ref.at[pl.ds(asn0 + i * 80, 80)], asn_t)
        pltpu.sync_copy(h_ref.at[pl.ds(adn0 + i * 80, 80)], adn_t)
        pltpu.sync_copy(zeros_ref.at[pl.ds(s * STRIPE, STRIPE)],
                        acc.at[pl.ds(s * STRIPE, STRIPE)])
        pltpu.sync_copy(zeros_ref.at[pl.ds(0, 8)], dacc.at[pl.ds(s * 8, 8)])
        plsc.subcore_barrier()

        def batch_body(j, _):
            e0 = (k0_s + 32 * j) * EB
            pltpu.sync_copy(esrc_ref.at[pl.ds(e0, EB)], src_v)
            pltpu.sync_copy(edst_ref.at[pl.ds(e0, EB)], dst_v)
            for g in range(EB // 16):
                sl = pl.ds(g * 16, 16)
                sg = src_v[sl]
                dg = dst_v[sl]
                a = plsc.load_gather(asn_t, [sg >> 7, sg & 127])
                b = plsc.load_gather(adn_t, [dg >> 7, dg & 127])
                e = a + b
                e = jnp.where(e >= 0, e, 0.2 * e)
                w = jnp.exp(e)
                eix = e0 + g * 16 + iota
                valid = (eix >= lo_s) & (eix < hi_s)
                w = jnp.where(valid, w, 0.0)
                lanes = dg & 127
                w_v[sl] = w
                didx_v[sl] = jnp.where(valid, dg, TRASH)
                dden_v[sl] = jnp.where(valid, dg >> 7, 120)
                lane_v[sl] = lanes
                src_v[sl] = sg + i * NPAD
                plsc.store_scatter(wrows_v, [g * 16 + iota, lanes], w)
            pltpu.async_copy(h_ref.at[src_v], rows_v, sem).wait()

            def row_body(r, _):
                wb = plsc.load_gather(w_v, [jnp.zeros((16,), jnp.int32) + r])
                for cc in range(D // 16):
                    csl = pl.ds(cc * 16, 16)
                    rows_v[r, csl] = rows_v[r, csl] * wb
                return 0

            lax.fori_loop(0, EB, row_body, 0, unroll=4)
            pltpu.sync_copy(rows_v, acc.at[didx_v], add=True)
            pltpu.sync_copy(wrows_v, dacc.at[dden_v], add=True)
            for g in range(EB // 16):
                sl = pl.ds(g * 16, 16)
                plsc.store_scatter(wrows_v, [g * 16 + iota, lane_v[sl]], zvec)
            return 0

        lax.fori_loop(0, nk_s, batch_body, 0)
        plsc.subcore_barrier()
        pltpu.sync_copy(acc.at[pl.ds(s * STRIPE, STRIPE)],
                        num_out.at[c, i, pl.ds(s * STRIPE, STRIPE)])
        pltpu.sync_copy(dacc.at[pl.ds(s * 8, 8)],
                        den_out.at[c, i, pl.ds(s * 8, 8)])
        return 0

    lax.fori_loop(0, n_agg, agg_body, 0)


def _edge_phase(H, asn, adn, edge_src, edge_dst, edge_offsets, n_agg):
    T = edge_src.shape[0]
    nbatch = -(-T // EB)
    tpad = nbatch * EB
    pad = jnp.zeros((tpad - T,), jnp.int32)
    esrc = jnp.concatenate([edge_src, pad])
    edst = jnp.concatenate([edge_dst, pad])
    off = edge_offsets.astype(jnp.int32)
    lo = off[:-1]
    hi = off[1:]
    b0 = lo // EB
    b1 = -(-hi // EB)
    cs = jnp.arange(2, dtype=jnp.int32)[:, None, None]
    ss = jnp.arange(16, dtype=jnp.int32)[None, :, None]
    k0 = b0[None, None, :] + cs + 2 * ss
    nk = jnp.maximum(0, -(-(b1[None, None, :] - k0) // 32))
    desc = jnp.zeros((2, 16, n_agg, 16), jnp.int32)
    desc = desc.at[..., 0].set(jnp.broadcast_to(lo, (2, 16, n_agg)))
    desc = desc.at[..., 1].set(jnp.broadcast_to(hi, (2, 16, n_agg)))
    desc = desc.at[..., 2].set(k0)
    desc = desc.at[..., 3].set(nk)
    desc = desc.reshape(2, 16, n_agg * 16)
    zeros_h = jnp.zeros((ROWS, D), jnp.float32)

    mesh = plsc.VectorSubcoreMesh(core_axis_name="c", subcore_axis_name="s")
    f = pl.kernel(
        functools.partial(_sc_body, n_agg),
        out_type=[
            jax.ShapeDtypeStruct((2, n_agg, ROWS, D), jnp.float32),
            jax.ShapeDtypeStruct((2, n_agg, DROWS, D), jnp.float32),
        ],
        mesh=mesh,
        compiler_params=pltpu.CompilerParams(needs_layout_passes=False),
        scratch_types=[
            pltpu.VMEM((80, D), jnp.float32),
            pltpu.VMEM((80, D), jnp.float32),
            pltpu.VMEM((n_agg * 16,), jnp.int32),
            pltpu.VMEM((EB,), jnp.int32),
            pltpu.VMEM((EB,), jnp.int32),
            pltpu.VMEM((EB,), jnp.int32),
            pltpu.VMEM((EB,), jnp.int32),
            pltpu.VMEM((EB,), jnp.int32),
            pltpu.VMEM((EB,), jnp.float32),
            pltpu.VMEM((EB, D), jnp.float32),
            pltpu.VMEM((EB, D), jnp.float32),
            pltpu.VMEM_SHARED((ROWS, D), jnp.float32),
            pltpu.VMEM_SHARED((DROWS, D), jnp.float32),
            pltpu.SemaphoreType.DMA,
        ],
    )
    def pack(a):
        return a.reshape(n_agg * 80, D)

    haug = jnp.concatenate([H.reshape(n_agg * NPAD, D), pack(asn), pack(adn)])
    num, den = f(haug, esrc, edst, desc, zeros_h)
    return num, den.reshape(2, n_agg, 1, DROWS * D)


def _tc2_body(x_ref, ws_ref, as_ref, ad_ref, num_ref, den_ref, bias_ref,
              w1x_ref, w1r_ref, b1_ref, w2_ref, b2_ref, y_ref, ident_ref, *,
              n_agg):
    i = pl.program_id(1)

    @pl.when(i == 0)
    def _():
        y_ref[...] = jnp.dot(x_ref[...], w1x_ref[...],
                             preferred_element_type=jnp.float32)
        r = lax.broadcasted_iota(jnp.int32, (NB, NB), 0)
        l = lax.broadcasted_iota(jnp.int32, (NB, NB), 1)
        ident_ref[...] = jnp.where(r == l, 1.0, 0.0)

    h = jnp.dot(x_ref[...], ws_ref[0], preferred_element_type=jnp.float32)
    asn = jnp.sum(h * as_ref[0], axis=1, keepdims=True)
    adn = jnp.sum(h * ad_ref[0], axis=1, keepdims=True)
    e_self = asn + adn
    e_self = jnp.where(e_self >= 0, e_self, 0.2 * e_self)
    es = jnp.exp(e_self)
    den_lane = den_ref[0, 0] + den_ref[1, 0]
    den = lax.dot_general(ident_ref[...], den_lane, (((1,), (1,)), ((), ())),
                          preferred_element_type=jnp.float32)
    num = jnp.sum(num_ref[:, 0], axis=0) + es * h
    out_i = num / (den + es) + bias_ref[0]
    y_ref[...] += jnp.dot(out_i, w1r_ref[0], preferred_element_type=jnp.float32)

    @pl.when(i == n_agg - 1)
    def _():
        mid = jnp.tanh(y_ref[...] + b1_ref[...])
        y_ref[...] = jnp.dot(mid, w2_ref[...],
                             preferred_element_type=jnp.float32) + b2_ref[...]


def _phase3(x, Ws, att_src, att_dst, num, den, bias_gat, W1, b1, W2, b2):
    n_agg = Ws.shape[0]
    W1x = W1[:D]
    W1r = W1[D:].reshape(n_agg, D, D)
    grid = (NBLK, n_agg)
    return pl.pallas_call(
        functools.partial(_tc2_body, n_agg=n_agg),
        grid=grid,
        in_specs=[
            pl.BlockSpec((NB, D), lambda b, i: (b, 0)),
            pl.BlockSpec((1, D, D), lambda b, i: (i, 0, 0)),
            pl.BlockSpec((1, 1, D), lambda b, i: (i, 0, 0)),
            pl.BlockSpec((1, 1, D), lambda b, i: (i, 0, 0)),
            pl.BlockSpec((2, 1, NB, D), lambda b, i: (0, i, b, 0)),
            pl.BlockSpec((2, 1, 1, NB), lambda b, i: (0, i, 0, b)),
            pl.BlockSpec((1, 1, D), lambda b, i: (i, 0, 0)),
            pl.BlockSpec((D, D), lambda b, i: (0, 0)),
            pl.BlockSpec((1, D, D), lambda b, i: (i, 0, 0)),
            pl.BlockSpec((1, D), lambda b, i: (0, 0)),
            pl.BlockSpec((D, D), lambda b, i: (0, 0)),
            pl.BlockSpec((1, D), lambda b, i: (0, 0)),
        ],
        out_specs=pl.BlockSpec((NB, D), lambda b, i: (b, 0)),
        out_shape=jax.ShapeDtypeStruct((NPAD, D), jnp.float32),
        scratch_shapes=[pltpu.VMEM((NB, NB), jnp.float32)],
        compiler_params=pltpu.CompilerParams(
            dimension_semantics=("parallel", "arbitrary")),
    )(x, Ws, att_src.reshape(n_agg, 1, D), att_dst.reshape(n_agg, 1, D),
      num, den, bias_gat.reshape(n_agg, 1, D), W1x, W1r,
      b1.reshape(1, D), W2, b2.reshape(1, D))


def kernel(x, Ws, att_src, att_dst, bias_gat, W1, b1, W2, b2,
           edge_src, edge_dst, edge_offsets):
    n_agg = Ws.shape[0]
    xp = jnp.concatenate([x, jnp.zeros((NPAD - N, D), jnp.float32)])
    H, asn, adn = _phase1(xp, Ws, att_src, att_dst)
    num, den = _edge_phase(H, asn, adn, edge_src.astype(jnp.int32),
                           edge_dst.astype(jnp.int32), edge_offsets, n_agg)
    y = _phase3(xp, Ws, att_src, att_dst, num, den, bias_gat, W1, b1, W2, b2)
    return y[:N]
